# baseline (speedup 1.0000x reference)
import functools

import jax
import jax.numpy as jnp
from jax import lax
from jax.experimental import pallas as pl
from jax.experimental.pallas import tpu as pltpu
from jax.experimental.pallas import tpu_sc as plsc

D = 1024
SCALE = 32.0
LANES = 16

NC = 2
NS = 16
NW = NC * NS

B_TOTAL = 4 * 8192
BPW = B_TOTAL // NW
C = 32
NCHUNK = BPW // C
NBUF = 3

_mesh = plsc.VectorSubcoreMesh(core_axis_name="c", subcore_axis_name="s")


@functools.partial(
    pl.kernel,
    mesh=_mesh,
    out_type=jax.ShapeDtypeStruct((B_TOTAL, D), jnp.float32),
    scratch_types=[
        pltpu.VMEM((NCHUNK, C), jnp.int32),
        pltpu.VMEM((NBUF, C, D), jnp.float32),
        pltpu.SemaphoreType.DMA,
        pltpu.SemaphoreType.DMA,
    ],
)
def _emb_lookup(x_hbm, table_hbm, out_hbm, idx_v, rows_v, semg, sems):
    wid = lax.axis_index("s") * NC + lax.axis_index("c")
    base = wid * BPW
    pltpu.sync_copy(x_hbm.at[wid], idx_v)

    def gather(g):
        return pltpu.async_copy(
            table_hbm.at[idx_v.at[g]], rows_v.at[g % NBUF], semg)

    gath = {0: gather(0), 1: gather(1)}
    scat = {}
    pending = []
    for g in range(NCHUNK):
        b = g % NBUF
        gath[g].wait()
        if g + 2 < NCHUNK:
            if g >= 1:
                scat[g - 1].wait()
                pending.remove(g - 1)
            gath[g + 2] = gather(g + 2)

        def row(r, carry, b=b):
            for c in range(D // LANES):
                sl = pl.ds(c * LANES, LANES)
                rows_v[b, r, sl] = rows_v[b, r, sl] * SCALE
            return carry

        lax.fori_loop(0, C, row, 0)
        scat[g] = pltpu.async_copy(
            rows_v.at[b], out_hbm.at[pl.ds(base + g * C, C)], sems)
        pending.append(g)
    for g in pending:
        scat[g].wait()


def kernel(x, table):
    xf = x.reshape(NW, NCHUNK, C)
    out = _emb_lookup(xf, table)
    return out.reshape(4, 8192, D)


# speedup vs baseline: 1.0375x; 1.0375x over previous
import functools

import jax
import jax.numpy as jnp
from jax import lax
from jax.experimental import pallas as pl
from jax.experimental.pallas import tpu as pltpu
from jax.experimental.pallas import tpu_sc as plsc

D = 1024
SCALE = 32.0
LANES = 16

NC = 2
NS = 16
NW = NC * NS

B_TOTAL = 4 * 8192
BPW = B_TOTAL // NW
C = 16
NCHUNK = BPW // C
NBUF = 4
LEAD = 2

_mesh = plsc.VectorSubcoreMesh(core_axis_name="c", subcore_axis_name="s")


@functools.partial(
    pl.kernel,
    mesh=_mesh,
    out_type=jax.ShapeDtypeStruct((B_TOTAL, D), jnp.float32),
    scratch_types=[
        pltpu.VMEM((NCHUNK, C), jnp.int32),
        pltpu.VMEM((NBUF, C, D), jnp.float32),
        pltpu.SemaphoreType.DMA,
        pltpu.SemaphoreType.DMA,
    ],
)
def _emb_lookup(x_hbm, table_hbm, out_hbm, idx_v, rows_v, semg, sems):
    wid = lax.axis_index("s") * NC + lax.axis_index("c")
    base = wid * BPW
    pltpu.sync_copy(x_hbm.at[wid], idx_v)

    def gather(q, b):
        return pltpu.async_copy(table_hbm.at[idx_v.at[q]], rows_v.at[b], semg)

    def scatter(q, b):
        return pltpu.async_copy(
            rows_v.at[b], out_hbm.at[pl.ds(base + q * C, C)], sems)

    def drain(sem, b):
        pltpu.make_async_copy(out_hbm.at[pl.ds(base, C)], rows_v.at[b], sem).wait()

    def scale(b):
        def row(r, carry):
            for c in range(D // LANES):
                sl = pl.ds(c * LANES, LANES)
                rows_v[b, r, sl] = rows_v[b, r, sl] * SCALE
            return carry
        lax.fori_loop(0, C, row, 0)

    g0, g1 = gather(0, 0), gather(1, 1)
    g0.wait()
    scale(0)
    scatter(0, 0)
    gather(2, 2)
    g1.wait()
    scale(1)
    scatter(1, 1)
    gather(3, 3)

    def group(j, carry):
        for i in range(NBUF):
            q = 2 + j * NBUF + i
            b = (2 + i) % NBUF
            drain(semg, b)
            scale(b)
            scatter(q, b)
            drain(sems, (b + LEAD) % NBUF)
            gather(q + LEAD, (b + LEAD) % NBUF)
        return carry

    lax.fori_loop(0, (NCHUNK - 2 - LEAD) // NBUF, group, 0)

    for q in (NCHUNK - 2, NCHUNK - 1):
        b = q % NBUF
        drain(semg, b)
        scale(b)
        scatter(q, b)
    for b in range(NBUF):
        drain(sems, b)


def kernel(x, table):
    xf = x.reshape(NW, NCHUNK, C)
    out = _emb_lookup(xf, table)
    return out.reshape(4, 8192, D)


# speedup vs baseline: 1.0888x; 1.0495x over previous
import functools

import jax
import jax.numpy as jnp
from jax import lax
from jax.experimental import pallas as pl
from jax.experimental.pallas import tpu as pltpu
from jax.experimental.pallas import tpu_sc as plsc

D = 1024
SCALE = 32.0
LANES = 16

NC = 2
NS = 16
NW = NC * NS

B_TOTAL = 4 * 8192
BPW = B_TOTAL // NW
C = 32
H = C // 2
NCHUNK = BPW // C
NBUF = 3

_mesh = plsc.VectorSubcoreMesh(core_axis_name="c", subcore_axis_name="s")


@functools.partial(
    pl.kernel,
    mesh=_mesh,
    out_type=jax.ShapeDtypeStruct((B_TOTAL, D), jnp.float32),
    scratch_types=[
        pltpu.VMEM((NCHUNK, C), jnp.int32),
        pltpu.VMEM((NBUF, C, D), jnp.float32),
        pltpu.SemaphoreType.DMA,
        pltpu.SemaphoreType.DMA,
    ],
)
def _emb_lookup(x_hbm, table_hbm, out_hbm, idx_v, rows_v, semg, sems):
    wid = lax.axis_index("s") * NC + lax.axis_index("c")
    base = wid * BPW
    pltpu.sync_copy(x_hbm.at[wid], idx_v)

    def gather(q, b):
        return pltpu.async_copy(table_hbm.at[idx_v.at[q]], rows_v.at[b], semg)

    def drain_gather(b):
        pltpu.make_async_copy(
            out_hbm.at[pl.ds(base, C)], rows_v.at[b], semg).wait()

    def drain_scatter(b):
        pltpu.make_async_copy(
            out_hbm.at[pl.ds(base, H)], rows_v.at[b].at[pl.ds(0, H)], sems
        ).wait()

    def scale_and_scatter(q, b):
        def row(r, carry):
            for c in range(D // LANES):
                sl = pl.ds(c * LANES, LANES)
                rows_v[b, r, sl] = rows_v[b, r, sl] * SCALE
            return carry

        for h in range(2):
            lax.fori_loop(h * H, (h + 1) * H, row, 0)
            pltpu.async_copy(
                rows_v.at[b].at[pl.ds(h * H, H)],
                out_hbm.at[pl.ds(base + q * C + h * H, H)], sems)

    g0, g1 = gather(0, 0), gather(1, 1)
    g0.wait()
    scale_and_scatter(0, 0)
    gather(2, 2)
    g1.wait()
    scale_and_scatter(1, 1)
    drain_scatter(0)
    drain_scatter(0)
    gather(3, 0)

    def group(j, carry):
        for i in range(NBUF):
            q = 2 + j * NBUF + i
            b = (2 + i) % NBUF
            drain_gather(b)
            scale_and_scatter(q, b)
            nb = (b + 2) % NBUF
            drain_scatter(nb)
            drain_scatter(nb)
            gather(q + 2, nb)
        return carry

    lax.fori_loop(0, (NCHUNK - 5) // NBUF, group, 0)

    q = NCHUNK - 3
    b = q % NBUF
    drain_gather(b)
    scale_and_scatter(q, b)
    nb = (b + 2) % NBUF
    drain_scatter(nb)
    drain_scatter(nb)
    gather(q + 2, nb)
    for q in (NCHUNK - 2, NCHUNK - 1):
        b = q % NBUF
        drain_gather(b)
        scale_and_scatter(q, b)
    for _ in range(3):
        drain_scatter(0)
        drain_scatter(0)


def kernel(x, table):
    xf = x.reshape(NW, NCHUNK, C)
    out = _emb_lookup(xf, table)
    return out.reshape(4, 8192, D)


# speedup vs baseline: 1.0968x; 1.0074x over previous
import functools

import jax
import jax.numpy as jnp
from jax import lax
from jax.experimental import pallas as pl
from jax.experimental.pallas import tpu as pltpu
from jax.experimental.pallas import tpu_sc as plsc

D = 1024
SCALE = 32.0
LANES = 16

NC = 2
NS = 16
NW = NC * NS

B_TOTAL = 4 * 8192
BPW = B_TOTAL // NW
C = 32
H = C // 2
NCHUNK = BPW // C
NBUF = 3

_mesh = plsc.VectorSubcoreMesh(core_axis_name="c", subcore_axis_name="s")


@functools.partial(
    pl.kernel,
    mesh=_mesh,
    out_type=jax.ShapeDtypeStruct((B_TOTAL, D), jnp.float32),
    scratch_types=[
        pltpu.VMEM((NCHUNK, C), jnp.int32),
        pltpu.VMEM((NBUF, C, D), jnp.float32),
        pltpu.SemaphoreType.DMA,
        pltpu.SemaphoreType.DMA,
    ],
)
def _emb_lookup(x_hbm, table_hbm, out_hbm, idx_v, rows_v, semg, sems):
    wid = lax.axis_index("s") * NC + lax.axis_index("c")
    base = wid * BPW
    pltpu.sync_copy(x_hbm.at[wid], idx_v)

    def gather(q, b):
        return pltpu.async_copy(table_hbm.at[idx_v.at[q]], rows_v.at[b], semg)

    def drain_gather(b):
        pltpu.make_async_copy(
            out_hbm.at[pl.ds(base, C)], rows_v.at[b], semg).wait()

    def drain_scatter(b):
        pltpu.make_async_copy(
            out_hbm.at[pl.ds(base, H)], rows_v.at[b].at[pl.ds(0, H)], sems
        ).wait()

    def scale_and_scatter(q, b, mid=None):
        def row(r, carry):
            for c in range(D // LANES):
                sl = pl.ds(c * LANES, LANES)
                rows_v[b, r, sl] = rows_v[b, r, sl] * SCALE
            return carry

        for h in range(2):
            lax.fori_loop(h * H, (h + 1) * H, row, 0)
            pltpu.async_copy(
                rows_v.at[b].at[pl.ds(h * H, H)],
                out_hbm.at[pl.ds(base + q * C + h * H, H)], sems)
            if h == 0 and mid is not None:
                mid()

    g0, g1 = gather(0, 0), gather(1, 1)
    g0.wait()
    scale_and_scatter(0, 0)
    gather(2, 2)
    g1.wait()
    scale_and_scatter(1, 1)
    drain_scatter(0)
    drain_scatter(0)
    gather(3, 0)

    def group(j, carry):
        for i in range(NBUF):
            q = 2 + j * NBUF + i
            b = (2 + i) % NBUF
            drain_gather(b)
            nb = (b + 2) % NBUF

            def mid(q=q, nb=nb):
                drain_scatter(nb)
                drain_scatter(nb)
                gather(q + 2, nb)

            scale_and_scatter(q, b, mid)
        return carry

    lax.fori_loop(0, (NCHUNK - 5) // NBUF, group, 0)

    q = NCHUNK - 3
    b = q % NBUF
    drain_gather(b)
    scale_and_scatter(q, b)
    nb = (b + 2) % NBUF
    drain_scatter(nb)
    drain_scatter(nb)
    gather(q + 2, nb)
    for q in (NCHUNK - 2, NCHUNK - 1):
        b = q % NBUF
        drain_gather(b)
        scale_and_scatter(q, b)
    for _ in range(3):
        drain_scatter(0)
        drain_scatter(0)


def kernel(x, table):
    xf = x.reshape(NW, NCHUNK, C)
    out = _emb_lookup(xf, table)
    return out.reshape(4, 8192, D)
